# baseline (speedup 1.0000x reference)
import jax
import jax.numpy as jnp
from jax import lax
from jax.experimental import pallas as pl
from jax.experimental.pallas import tpu as pltpu
from jax.experimental.pallas import tpu_sc as plsc

N_VOCAB = 100000
N_EMBD = 64
N_TOKEN = 200
BATCH = 4096

NC = 2
NS = 16
K = N_TOKEN // NC
BBLK = BATCH // NS
LANES = 16
VPR = N_EMBD // LANES
NBUF = 2


def _emb_kernel(table_hbm, idx_hbm, pos_hbm, out_hbm,
                idxb, pos_v, bufs, buf2, shared, isems, gsems, lsem, wsems):
    cidx = lax.axis_index("c")
    sid = lax.axis_index("s")
    col0 = sid * BBLK

    pltpu.sync_copy(pos_hbm, pos_v)

    def stage_idx(k, b):
        return pltpu.make_async_copy(
            idx_hbm.at[cidx, k, pl.ds(col0, BBLK)], idxb[b], isems[b])

    def gather(k, b):
        del k
        return [pltpu.make_async_copy(
            table_hbm.at[idxb[b].at[pl.ds(h * 128, 128)]],
            bufs[b].at[pl.ds(h * 128, 128)], gsems[b]) for h in range(2)]

    def write(k, b):
        t = NC * k + cidx
        return pltpu.make_async_copy(shared.at[b], out_hbm.at[t], wsems[b])

    for b in range(NBUF):
        stage_idx(b, b).start()
        stage_idx(b, b).wait()
        for cp in gather(b, b):
            cp.start()

    rows = [c * LANES + lax.iota(jnp.int32, LANES) for c in range(VPR)]

    def outer(k2, carry):
        for b in range(NBUF):
            k = k2 * NBUF + b
            t = NC * k + cidx
            for cp in gather(k, b):
                cp.wait()

            @pl.when(k + NBUF < K)
            def _():
                stage_idx(k + NBUF, b).start()

            pv = [pos_v[t, pl.ds(c * LANES, LANES)] for c in range(VPR)]

            def tr_body(bb, c2):
                cols = jnp.full((LANES,), bb, jnp.int32)
                for c in range(VPR):
                    v = bufs[b][bb, pl.ds(c * LANES, LANES)] + pv[c]
                    plsc.store_scatter(buf2, [rows[c], cols], v)
                return c2
            lax.fori_loop(0, BBLK, tr_body, 0, unroll=2)

            @pl.when(k + NBUF < K)
            def _():
                stage_idx(k + NBUF, b).wait()
                for cp in gather(k + NBUF, b):
                    cp.start()

            @pl.when(jnp.logical_and(sid == 0, k >= NBUF))
            def _():
                write(k - NBUF, b).wait()
            plsc.subcore_barrier()

            cpl = pltpu.make_async_copy(
                buf2, shared.at[b, :, pl.ds(col0, BBLK)], lsem)
            cpl.start()
            cpl.wait()
            plsc.subcore_barrier()

            @pl.when(sid == 0)
            def _():
                write(k, b).start()
        return carry

    lax.fori_loop(0, K // NBUF, outer, 0)

    @pl.when(sid == 0)
    def _():
        for b in range(NBUF):
            write(K - NBUF + b, b).wait()
    plsc.subcore_barrier()


def _emb(table, idx_arr, pos):
    mesh = plsc.VectorSubcoreMesh(core_axis_name="c", subcore_axis_name="s")
    f = pl.kernel(
        _emb_kernel,
        out_type=jax.ShapeDtypeStruct((N_TOKEN, N_EMBD, BATCH), jnp.float32),
        mesh=mesh,
        scratch_types=[
            [pltpu.VMEM((BBLK,), jnp.int32) for _ in range(NBUF)],
            pltpu.VMEM((N_TOKEN, N_EMBD), jnp.float32),
            [pltpu.VMEM((BBLK, N_EMBD), jnp.float32) for _ in range(NBUF)],
            pltpu.VMEM((N_EMBD, BBLK), jnp.float32),
            pltpu.VMEM_SHARED((NBUF, N_EMBD, BATCH), jnp.float32),
            [pltpu.SemaphoreType.DMA for _ in range(NBUF)],
            [pltpu.SemaphoreType.DMA for _ in range(NBUF)],
            pltpu.SemaphoreType.DMA,
            [pltpu.SemaphoreType.DMA for _ in range(NBUF)],
        ],
        compiler_params=pltpu.CompilerParams(
            use_tc_tiling_on_sc=False, needs_layout_passes=False),
    )
    return f(table, idx_arr, pos)


def kernel(tokens, token_embedding, position_embedding):
    idx_arr = tokens.T.reshape(K, NC, BATCH).transpose(1, 0, 2)
    out_t = _emb(token_embedding, idx_arr, position_embedding)
    return out_t.transpose(2, 0, 1)


# speedup vs baseline: 1.2394x; 1.2394x over previous
import jax
import jax.numpy as jnp
from jax import lax
from jax.experimental import pallas as pl
from jax.experimental.pallas import tpu as pltpu
from jax.experimental.pallas import tpu_sc as plsc

N_VOCAB = 100000
N_EMBD = 64
N_TOKEN = 200
BATCH = 4096

NC = 2
NS = 16
NW = NC * NS
BBLK = BATCH // NW
LANES = 16
VPR = N_EMBD // LANES
TROWS = N_TOKEN * N_EMBD // 8
NBUF = 4


def _emb_kernel(table_hbm, idx_hbm, pos_hbm, out_hbm,
                idx_v, pos_v, bufs, buf2s, gsems, wsems):
    wid = lax.axis_index("s") * NC + lax.axis_index("c")
    col0 = wid * BBLK

    pltpu.sync_copy(idx_hbm.at[:, pl.ds(col0, BBLK)], idx_v)
    pltpu.sync_copy(pos_hbm, pos_v)

    def gather(t, b):
        return pltpu.make_async_copy(
            table_hbm.at[idx_v.at[t]], bufs[b], gsems[b])

    def write(t, b):
        return pltpu.make_async_copy(
            buf2s[b], out_hbm.at[pl.ds(8 * t, 8), wid], wsems[b])

    for b in range(NBUF):
        gather(b, b).start()

    iota = lax.iota(jnp.int32, LANES)
    evec = [c * LANES + iota for c in range(VPR)]
    eo_i = [e >> 3 for e in evec]
    r_i = [e & 7 for e in evec]

    def outer(k, carry):
        i = k * NBUF
        for b in range(NBUF):
            t = i + b
            gather(t, b).wait()

            @pl.when(t >= NBUF)
            def _():
                write(t - NBUF, b).wait()

            pv = [pos_v[t, pl.ds(c * LANES, LANES)] for c in range(VPR)]

            def tr_body(bb, c2):
                cols = jnp.full((LANES,), bb, jnp.int32)
                for c in range(VPR):
                    v = bufs[b][bb, pl.ds(c * LANES, LANES)] + pv[c]
                    plsc.store_scatter(buf2s[b], [eo_i[c], r_i[c], cols], v)
                return c2
            lax.fori_loop(0, BBLK, tr_body, 0, unroll=2)

            write(t, b).start()

            @pl.when(t + NBUF < N_TOKEN)
            def _():
                gather(t + NBUF, b).start()
        return carry

    lax.fori_loop(0, N_TOKEN // NBUF, outer, 0)

    for b in range(NBUF):
        write(N_TOKEN - NBUF + b, b).wait()


def _emb(table, idx_t, pos):
    mesh = plsc.VectorSubcoreMesh(core_axis_name="c", subcore_axis_name="s")
    f = pl.kernel(
        _emb_kernel,
        out_type=jax.ShapeDtypeStruct((TROWS, NW, 8, 128), jnp.float32),
        mesh=mesh,
        scratch_types=[
            pltpu.VMEM((N_TOKEN, BBLK), jnp.int32),
            pltpu.VMEM((N_TOKEN, N_EMBD), jnp.float32),
            [pltpu.VMEM((BBLK, N_EMBD), jnp.float32) for _ in range(NBUF)],
            [pltpu.VMEM((8, 8, BBLK), jnp.float32) for _ in range(NBUF)],
            [pltpu.SemaphoreType.DMA for _ in range(NBUF)],
            [pltpu.SemaphoreType.DMA for _ in range(NBUF)],
        ],
        compiler_params=pltpu.CompilerParams(
            use_tc_tiling_on_sc=False, needs_layout_passes=False),
    )
    return f(table, idx_t, pos)


def kernel(tokens, token_embedding, position_embedding):
    idx_t = tokens.T
    out4 = _emb(token_embedding, idx_t, position_embedding)
    y = out4.reshape(N_TOKEN, 8, NW, 8, 128).transpose(2, 4, 0, 1, 3)
    return y.reshape(BATCH, N_TOKEN, N_EMBD)


# speedup vs baseline: 2.0414x; 1.6471x over previous
import jax
import jax.numpy as jnp
from jax import lax
from jax.experimental import pallas as pl
from jax.experimental.pallas import tpu as pltpu
from jax.experimental.pallas import tpu_sc as plsc

N_VOCAB = 100000
N_EMBD = 64
N_TOKEN = 200
BATCH = 4096

NC = 2
NS = 16
NW = NC * NS
BBLK = BATCH // NW
LANES = 16
VPR = N_EMBD // LANES
TROWS = N_TOKEN * N_EMBD // 8
NBUF = 4


def _emb_kernel(table_hbm, idx_hbm, pos_hbm, out_hbm,
                idx_v, pos_v, bufs, buf2s, gsems, wsems):
    wid = lax.axis_index("s") * NC + lax.axis_index("c")
    col0 = wid * BBLK

    pltpu.sync_copy(idx_hbm.at[:, pl.ds(col0, BBLK)], idx_v)
    pltpu.sync_copy(pos_hbm, pos_v)

    def gather(t, b):
        return pltpu.make_async_copy(
            table_hbm.at[idx_v.at[t]], bufs[b], gsems[b])

    def write(t, b):
        return pltpu.make_async_copy(
            buf2s[b], out_hbm.at[pl.ds(8 * t, 8), wid], wsems[b])

    for b in range(NBUF):
        gather(b, b).start()

    iota = lax.iota(jnp.int32, LANES)

    def outer(k, carry):
        i = k * NBUF
        for b in range(NBUF):
            t = i + b
            gather(t, b).wait()

            @pl.when(t >= NBUF)
            def _():
                write(t - NBUF, b).wait()

            pv = [pos_v[t, pl.ds(c * LANES, LANES)] for c in range(VPR)]

            def add_body(bb, c2):
                for c in range(VPR):
                    sl = pl.ds(c * LANES, LANES)
                    bufs[b][bb, sl] = bufs[b][bb, sl] + pv[c]
                return c2
            lax.fori_loop(0, BBLK, add_body, 0, unroll=2)

            def tr_body(blk, c2):
                rows = blk * LANES + iota
                for c in range(VPR):
                    for d in range(LANES):
                        ecol = c * LANES + ((d + iota) & (LANES - 1))
                        v = plsc.load_gather(bufs[b], [rows, ecol])
                        plsc.store_scatter(
                            buf2s[b], [ecol >> 3, ecol & 7, rows], v)
                return c2
            lax.fori_loop(0, BBLK // LANES, tr_body, 0)

            write(t, b).start()

            @pl.when(t + NBUF < N_TOKEN)
            def _():
                gather(t + NBUF, b).start()
        return carry

    lax.fori_loop(0, N_TOKEN // NBUF, outer, 0)

    for b in range(NBUF):
        write(N_TOKEN - NBUF + b, b).wait()


def _emb(table, idx_t, pos):
    mesh = plsc.VectorSubcoreMesh(core_axis_name="c", subcore_axis_name="s")
    f = pl.kernel(
        _emb_kernel,
        out_type=jax.ShapeDtypeStruct((TROWS, NW, 8, 128), jnp.float32),
        mesh=mesh,
        scratch_types=[
            pltpu.VMEM((N_TOKEN, BBLK), jnp.int32),
            pltpu.VMEM((N_TOKEN, N_EMBD), jnp.float32),
            [pltpu.VMEM((BBLK, N_EMBD), jnp.float32) for _ in range(NBUF)],
            [pltpu.VMEM((8, 8, BBLK), jnp.float32) for _ in range(NBUF)],
            [pltpu.SemaphoreType.DMA for _ in range(NBUF)],
            [pltpu.SemaphoreType.DMA for _ in range(NBUF)],
        ],
        compiler_params=pltpu.CompilerParams(
            use_tc_tiling_on_sc=False, needs_layout_passes=False),
    )
    return f(table, idx_t, pos)


def kernel(tokens, token_embedding, position_embedding):
    idx_t = tokens.T
    out4 = _emb(token_embedding, idx_t, position_embedding)
    y = out4.reshape(N_TOKEN, 8, NW, 8, 128).transpose(2, 4, 0, 1, 3)
    return y.reshape(BATCH, N_TOKEN, N_EMBD)


# speedup vs baseline: 2.4866x; 1.2181x over previous
import jax
import jax.numpy as jnp
from jax import lax
from jax.experimental import pallas as pl
from jax.experimental.pallas import tpu as pltpu
from jax.experimental.pallas import tpu_sc as plsc

N_VOCAB = 100000
N_EMBD = 64
N_TOKEN = 200
BATCH = 4096

NC = 2
NS = 16
NW = NC * NS
BBLK = BATCH // NW
LANES = 16
VPR = N_EMBD // LANES
TROWS = N_TOKEN * N_EMBD // 8
NBUF = 4


def _emb_kernel(table_hbm, idx_hbm, pos_hbm, out_hbm,
                idx_v, pos_v, bufs, buf2s, gsems, wsems):
    wid = lax.axis_index("s") * NC + lax.axis_index("c")
    col0 = wid * BBLK

    pltpu.sync_copy(idx_hbm.at[:, pl.ds(col0, BBLK)], idx_v)
    pltpu.sync_copy(pos_hbm, pos_v)

    def gather(t, b):
        return pltpu.make_async_copy(
            table_hbm.at[idx_v.at[t]], bufs[b], gsems[b])

    def write(t, b):
        return pltpu.make_async_copy(
            buf2s[b], out_hbm.at[pl.ds(8 * t, 8), wid], wsems[b])

    for b in range(NBUF):
        gather(b, b).start()

    iota = lax.iota(jnp.int32, LANES)

    def outer(k, carry):
        i = k * NBUF
        for b in range(NBUF):
            t = i + b
            gather(t, b).wait()

            @pl.when(t >= NBUF)
            def _():
                write(t - NBUF, b).wait()

            pv = [pos_v[t, pl.ds(c * LANES, LANES)] for c in range(VPR)]

            def tr_body(blk, c2):
                rows = blk * LANES + iota
                for c in range(VPR):
                    for d in range(LANES):
                        perm = (d + iota) & (LANES - 1)
                        ecol = c * LANES + perm
                        pvr = pv[c].at[perm].get(mode="promise_in_bounds")
                        v = plsc.load_gather(bufs[b], [rows, ecol]) + pvr
                        plsc.store_scatter(
                            buf2s[b], [ecol >> 3, ecol & 7, rows], v)
                return c2
            lax.fori_loop(0, BBLK // LANES, tr_body, 0)

            write(t, b).start()

            @pl.when(t + NBUF < N_TOKEN)
            def _():
                gather(t + NBUF, b).start()
        return carry

    lax.fori_loop(0, N_TOKEN // NBUF, outer, 0)

    for b in range(NBUF):
        write(N_TOKEN - NBUF + b, b).wait()


def _emb(table, idx_t, pos):
    mesh = plsc.VectorSubcoreMesh(core_axis_name="c", subcore_axis_name="s")
    f = pl.kernel(
        _emb_kernel,
        out_type=jax.ShapeDtypeStruct((TROWS, NW, 8, 128), jnp.float32),
        mesh=mesh,
        scratch_types=[
            pltpu.VMEM((N_TOKEN, BBLK), jnp.int32),
            pltpu.VMEM((N_TOKEN, N_EMBD), jnp.float32),
            [pltpu.VMEM((BBLK, N_EMBD), jnp.float32) for _ in range(NBUF)],
            [pltpu.VMEM((8, 8, BBLK), jnp.float32) for _ in range(NBUF)],
            [pltpu.SemaphoreType.DMA for _ in range(NBUF)],
            [pltpu.SemaphoreType.DMA for _ in range(NBUF)],
        ],
        compiler_params=pltpu.CompilerParams(
            use_tc_tiling_on_sc=False, needs_layout_passes=False),
    )
    return f(table, idx_t, pos)


def kernel(tokens, token_embedding, position_embedding):
    idx_t = tokens.T
    out4 = _emb(token_embedding, idx_t, position_embedding)
    y = out4.reshape(N_TOKEN, 8, NW, 8, 128).transpose(2, 4, 0, 1, 3)
    return y.reshape(BATCH, N_TOKEN, N_EMBD)
